# Initial kernel scaffold; baseline (speedup 1.0000x reference)
import jax
import jax.numpy as jnp
from jax.experimental import pallas as pl


def kernel(k_cache, v_cache, layer_idx, input_pos, k_val, v_val):
    raise NotImplementedError("write your pallas kernel here")


# speedup vs baseline: 3.0568x; 3.0568x over previous
import functools

import jax
import jax.numpy as jnp
from jax import lax
from jax.experimental import pallas as pl
from jax.experimental.pallas import tpu as pltpu
from jax.experimental.pallas import tpu_sc as plsc

N_LAYER, B, H, S, D = 4, 8, 8, 2048, 128
ROWS = B * H * S
NW = 32
RPW = ROWS // NW
CH = 256
NCH = RPW // CH
BH_PER_W = (B * H) // NW


def _sc_body(kc, vc, kval, vval, params_h, k_out, v_out,
             pbuf, buf, rbuf, sem):
    w = lax.axis_index("s") * 2 + lax.axis_index("c")
    pltpu.sync_copy(params_h, pbuf)
    pvec = pbuf[...]
    layer_base = pl.multiple_of(pvec[0], 8)
    pos = pvec[1]
    pos_div = pos // CH
    pos_mod = pos % CH
    base = w * RPW
    for src, dst, val in ((kc, k_out, kval), (vc, v_out, vval)):
        pltpu.sync_copy(val.at[pl.ds(w * BH_PER_W, BH_PER_W), :], rbuf)
        vrows = [[rbuf[j, pl.ds(16 * k, 16)] for k in range(D // 16)]
                 for j in range(BH_PER_W)]
        for c in range(NCH):
            r = base + c * CH
            pltpu.sync_copy(src.at[pl.ds(layer_base + r, CH), :], buf)
            for j in range(BH_PER_W):
                @pl.when(c == j * (S // CH) + pos_div)
                def _():
                    for k in range(D // 16):
                        buf[pos_mod, pl.ds(16 * k, 16)] = vrows[j][k]
            pltpu.sync_copy(buf, dst.at[pl.ds(r, CH), :])


@jax.jit
def _sc_update(kc2, vc2, kval2, vval2, params):
    f = pl.kernel(
        _sc_body,
        out_type=(jax.ShapeDtypeStruct((ROWS, D), jnp.float32),
                  jax.ShapeDtypeStruct((ROWS, D), jnp.float32)),
        mesh=plsc.VectorSubcoreMesh(core_axis_name="c", subcore_axis_name="s"),
        scratch_types=(
            pltpu.VMEM((16,), jnp.int32),
            pltpu.VMEM((CH, D), jnp.float32),
            pltpu.VMEM((BH_PER_W, D), jnp.float32),
            pltpu.SemaphoreType.DMA,
        ),
    )
    return f(kc2, vc2, kval2, vval2, params)


def kernel(k_cache, v_cache, layer_idx, input_pos, k_val, v_val):
    layer_idx = jnp.asarray(layer_idx, jnp.int32)
    input_pos = jnp.asarray(input_pos, jnp.int32)
    kc2 = k_cache.reshape(N_LAYER * ROWS, D)
    vc2 = v_cache.reshape(N_LAYER * ROWS, D)
    kval2 = k_val.reshape(B * H, D)
    vval2 = v_val.reshape(B * H, D)
    params = jnp.zeros((16,), jnp.int32)
    params = params.at[0].set(layer_idx * ROWS).at[1].set(input_pos)
    k2, v2 = _sc_update(kc2, vc2, kval2, vval2, params)
    return (k2.reshape(B, H, S, D), v2.reshape(B, H, S, D))


# speedup vs baseline: 3.6049x; 1.1793x over previous
import functools

import jax
import jax.numpy as jnp
from jax import lax
from jax.experimental import pallas as pl
from jax.experimental.pallas import tpu as pltpu
from jax.experimental.pallas import tpu_sc as plsc

N_LAYER, B, H, S, D = 4, 8, 8, 2048, 128
ROWS = B * H * S
NW = 32
RPW = ROWS // NW
CH = 256
NCH = RPW // CH
BH_PER_W = (B * H) // NW


def _sc_body(kc, vc, kval, vval, params_h, k_out, v_out,
             pbuf, bufs, rbuf, gsems, ssems):
    w = lax.axis_index("s") * 2 + lax.axis_index("c")
    pltpu.sync_copy(params_h, pbuf)
    pvec = pbuf[...]
    layer_base = pl.multiple_of(pvec[0], 8)
    pos = pvec[1]
    pos_div = pos // CH
    pos_mod = pos % CH
    base = w * RPW

    pltpu.sync_copy(kval.at[pl.ds(w * BH_PER_W, BH_PER_W), :], rbuf.at[0])
    pltpu.sync_copy(vval.at[pl.ds(w * BH_PER_W, BH_PER_W), :], rbuf.at[1])
    vrows = [[[rbuf[t, j, pl.ds(16 * k, 16)] for k in range(D // 16)]
              for j in range(BH_PER_W)]
             for t in range(2)]

    work = ([(0, kc, k_out, c) for c in range(NCH)] +
            [(1, vc, v_out, c) for c in range(NCH)])
    n = len(work)

    def gather(i, slot):
        _, src, _, c = work[i]
        r = base + c * CH
        return pltpu.make_async_copy(
            src.at[pl.ds(layer_base + r, CH), :], bufs.at[slot], gsems.at[slot])

    def scatter(i, slot):
        _, _, dst, c = work[i]
        r = base + c * CH
        return pltpu.make_async_copy(
            bufs.at[slot], dst.at[pl.ds(r, CH), :], ssems.at[slot])

    gather(0, 0).start()
    for i in range(n):
        slot = i % 2
        t, _, _, c = work[i]
        gather(i, slot).wait()
        for j in range(BH_PER_W):
            @pl.when(c == j * (S // CH) + pos_div)
            def _():
                for k in range(D // 16):
                    bufs[slot, pos_mod, pl.ds(16 * k, 16)] = vrows[t][j][k]
        scatter(i, slot).start()
        if i + 1 < n:
            nslot = (i + 1) % 2
            if i >= 1:
                scatter(i - 1, nslot).wait()
            gather(i + 1, nslot).start()
    scatter(n - 1, (n - 1) % 2).wait()


@jax.jit
def _sc_update(kc2, vc2, kval2, vval2, params):
    f = pl.kernel(
        _sc_body,
        out_type=(jax.ShapeDtypeStruct((ROWS, D), jnp.float32),
                  jax.ShapeDtypeStruct((ROWS, D), jnp.float32)),
        mesh=plsc.VectorSubcoreMesh(core_axis_name="c", subcore_axis_name="s"),
        scratch_types=(
            pltpu.VMEM((16,), jnp.int32),
            pltpu.VMEM((2, CH, D), jnp.float32),
            pltpu.VMEM((2, BH_PER_W, D), jnp.float32),
            pltpu.SemaphoreType.DMA((2,)),
            pltpu.SemaphoreType.DMA((2,)),
        ),
    )
    return f(kc2, vc2, kval2, vval2, params)


def kernel(k_cache, v_cache, layer_idx, input_pos, k_val, v_val):
    layer_idx = jnp.asarray(layer_idx, jnp.int32)
    input_pos = jnp.asarray(input_pos, jnp.int32)
    kc2 = k_cache.reshape(N_LAYER * ROWS, D)
    vc2 = v_cache.reshape(N_LAYER * ROWS, D)
    kval2 = k_val.reshape(B * H, D)
    vval2 = v_val.reshape(B * H, D)
    params = jnp.zeros((16,), jnp.int32)
    params = params.at[0].set(layer_idx * ROWS).at[1].set(input_pos)
    k2, v2 = _sc_update(kc2, vc2, kval2, vval2, params)
    return (k2.reshape(B, H, S, D), v2.reshape(B, H, S, D))


# speedup vs baseline: 3.6293x; 1.0067x over previous
import functools

import jax
import jax.numpy as jnp
from jax import lax
from jax.experimental import pallas as pl
from jax.experimental.pallas import tpu as pltpu
from jax.experimental.pallas import tpu_sc as plsc

N_LAYER, B, H, S, D = 4, 8, 8, 2048, 128
ROWS = B * H * S
NW = 32
RPW = ROWS // NW
CH = 256
NBUF = 3
NCH = RPW // CH
BH_PER_W = (B * H) // NW


def _sc_body(kc, vc, kval, vval, params_h, k_out, v_out,
             pbuf, bufs, rbuf, gsems, ssems):
    w = lax.axis_index("s") * 2 + lax.axis_index("c")
    pltpu.sync_copy(params_h, pbuf)
    pvec = pbuf[...]
    layer_base = pl.multiple_of(pvec[0], 8)
    pos = pvec[1]
    pos_div = pos // CH
    pos_mod = pos % CH
    base = w * RPW

    pltpu.sync_copy(kval.at[pl.ds(w * BH_PER_W, BH_PER_W), :], rbuf.at[0])
    pltpu.sync_copy(vval.at[pl.ds(w * BH_PER_W, BH_PER_W), :], rbuf.at[1])
    vrows = [[[rbuf[t, j, pl.ds(16 * k, 16)] for k in range(D // 16)]
              for j in range(BH_PER_W)]
             for t in range(2)]

    work = ([(0, kc, k_out, c) for c in range(NCH)] +
            [(1, vc, v_out, c) for c in range(NCH)])
    n = len(work)

    def gather(i, slot):
        _, src, _, c = work[i]
        r = base + c * CH
        return pltpu.make_async_copy(
            src.at[pl.ds(layer_base + r, CH), :], bufs.at[slot], gsems.at[slot])

    def scatter(i, slot):
        _, _, dst, c = work[i]
        r = base + c * CH
        return pltpu.make_async_copy(
            bufs.at[slot], dst.at[pl.ds(r, CH), :], ssems.at[slot])

    for p in range(NBUF - 1):
        gather(p, p).start()
    for i in range(n):
        slot = i % NBUF
        t, _, _, c = work[i]
        gather(i, slot).wait()
        for j in range(BH_PER_W):
            @pl.when(c == j * (S // CH) + pos_div)
            def _():
                for k in range(D // 16):
                    bufs[slot, pos_mod, pl.ds(16 * k, 16)] = vrows[t][j][k]
        scatter(i, slot).start()
        nxt = i + NBUF - 1
        if nxt < n:
            nslot = nxt % NBUF
            if nxt >= NBUF:
                scatter(nxt - NBUF, nslot).wait()
            gather(nxt, nslot).start()
    for i in range(max(0, n - NBUF), n):
        scatter(i, i % NBUF).wait()


@jax.jit
def _sc_update(kc2, vc2, kval2, vval2, params):
    f = pl.kernel(
        _sc_body,
        out_type=(jax.ShapeDtypeStruct((ROWS, D), jnp.float32),
                  jax.ShapeDtypeStruct((ROWS, D), jnp.float32)),
        mesh=plsc.VectorSubcoreMesh(core_axis_name="c", subcore_axis_name="s"),
        scratch_types=(
            pltpu.VMEM((16,), jnp.int32),
            pltpu.VMEM((NBUF, CH, D), jnp.float32),
            pltpu.VMEM((2, BH_PER_W, D), jnp.float32),
            pltpu.SemaphoreType.DMA((NBUF,)),
            pltpu.SemaphoreType.DMA((NBUF,)),
        ),
    )
    return f(kc2, vc2, kval2, vval2, params)


def kernel(k_cache, v_cache, layer_idx, input_pos, k_val, v_val):
    layer_idx = jnp.asarray(layer_idx, jnp.int32)
    input_pos = jnp.asarray(input_pos, jnp.int32)
    kc2 = k_cache.reshape(N_LAYER * ROWS, D)
    vc2 = v_cache.reshape(N_LAYER * ROWS, D)
    kval2 = k_val.reshape(B * H, D)
    vval2 = v_val.reshape(B * H, D)
    params = jnp.zeros((16,), jnp.int32)
    params = params.at[0].set(layer_idx * ROWS).at[1].set(input_pos)
    k2, v2 = _sc_update(kc2, vc2, kval2, vval2, params)
    return (k2.reshape(B, H, S, D), v2.reshape(B, H, S, D))
